# baseline (speedup 1.0000x reference)
import functools

import jax
import jax.numpy as jnp
from jax.experimental import pallas as pl

B, S, H = 8, 512, 1024
HCM = 512
SLOTS = 64
ALPHA = 0.1
GAMMA = 0.99
ETA = 0.05
BB = 2

_BF = jnp.bfloat16
_F32 = jnp.float32


def _fused_kernel(hs_ref, fast_ref, slow_ref,
                  w_item_b, b_item_ref, w_query_b, b_query_ref,
                  w_r1_b, b_r1_ref, w_r2_b, b_r2_ref,
                  w_mq_b, b_mq_ref,
                  wg_h_b, wg_r_b, b_g_ref,
                  wo_q_b, wo_r_b, b_o_ref,
                  ln_g_ref, ln_b_ref,
                  out_ref, newfast_ref, newslow_ref):
    x2 = hs_ref[...].reshape(BB * S, H)
    xb = x2.astype(_BF)

    items2 = jnp.dot(xb, w_item_b[...], preferred_element_type=_F32) + b_item_ref[...]
    h12 = jax.nn.gelu(jnp.dot(items2.astype(_BF), w_r1_b[...], preferred_element_type=_F32) + b_r1_ref[...])
    ub2 = jnp.dot(h12.astype(_BF), w_r2_b[...], preferred_element_type=_F32) + b_r2_ref[...]
    query2 = jnp.dot(xb, w_query_b[...], preferred_element_type=_F32) + b_query_ref[...]
    q_mem2 = jnp.dot(query2.astype(_BF), w_mq_b[...], preferred_element_type=_F32) + b_mq_ref[...]
    qk2 = ub2 * q_mem2

    scale = 1.0 / jnp.sqrt(jnp.float32(HCM))
    retr = []
    for i in range(BB):
        mem_b = jnp.concatenate([fast_ref[i], slow_ref[i]], axis=0).astype(_BF)
        qk = qk2[i * S:(i + 1) * S]
        scores = jax.lax.dot_general(qk.astype(_BF), mem_b, (((1,), (1,)), ((), ())),
                                     preferred_element_type=_F32) * scale
        m = jnp.max(scores, axis=-1, keepdims=True)
        p = jnp.exp(scores - m)
        w = p / jnp.sum(p, axis=-1, keepdims=True)
        retr.append(jnp.dot(w.astype(_BF), mem_b, preferred_element_type=_F32))
    retrieved2 = jnp.concatenate(retr, axis=0) * ub2
    retr_b = retrieved2.astype(_BF)

    fg2 = jax.nn.sigmoid(jnp.dot(xb, wg_h_b[...], preferred_element_type=_F32)
                         + jnp.dot(retr_b, wg_r_b[...], preferred_element_type=_F32)
                         + b_g_ref[...])

    t = jax.lax.broadcasted_iota(jnp.int32, (S, 1), 0)
    cons = (t % 10) == 0
    e = jnp.where(cons, 1.0 - ETA, 1.0)
    row = jax.lax.broadcasted_iota(jnp.int32, (S, S), 0)
    col = jax.lax.broadcasted_iota(jnp.int32, (S, S), 1)
    lower = (col <= row).astype(_F32)
    nafter = (S - 1) // 10 - t // 10
    wv = jnp.where(cons, (ETA / (1.0 - ETA)) * jnp.exp(nafter.astype(_F32) * jnp.log(_F32(GAMMA))), 0.0)
    ncons = (S + 9) // 10

    for i in range(BB):
        fast0 = fast_ref[i]
        slow0 = slow_ref[i]
        items = items2[i * S:(i + 1) * S]
        u = ALPHA * fg2[i * S:(i + 1) * S]
        logA = jnp.log((1.0 - u) * e)
        L = jnp.dot(lower, logA, preferred_element_type=_F32)
        Llast = L[S - 1:S, :]
        ue = u * e
        Cf = ue * jnp.exp(Llast - L)
        qv = wv * jnp.exp(L)
        wsum = jax.lax.dot_general(lower, qv, (((0,), (0,)), ((), ())),
                                   preferred_element_type=_F32)
        Cs = ue * wsum * jnp.exp(-L)
        plast_col = jnp.transpose(jnp.exp(Llast))
        w0_col = jnp.transpose(wsum[0:1, :])
        newfast_ref[i] = plast_col * fast0 + jax.lax.dot_general(
            Cf, items, (((0,), (0,)), ((), ())), preferred_element_type=_F32)
        newslow_ref[i] = (GAMMA ** ncons) * slow0 + w0_col * fast0 + jax.lax.dot_general(
            Cs, items, (((0,), (0,)), ((), ())), preferred_element_type=_F32)

    out2 = (jnp.dot(query2.astype(_BF), wo_q_b[...], preferred_element_type=_F32)
            + jnp.dot(retr_b, wo_r_b[...], preferred_element_type=_F32)
            + b_o_ref[...])
    y = x2 + out2
    mu = jnp.mean(y, axis=-1, keepdims=True)
    var = jnp.mean((y - mu) ** 2, axis=-1, keepdims=True)
    out_ref[...] = ((y - mu) / jnp.sqrt(var + 1e-5) * ln_g_ref[...] + ln_b_ref[...]).reshape(BB, S, H)


@functools.partial(jax.jit, static_argnames=())
def kernel(hidden_states, fast_hcm_state, slow_hcm_state, W_item, b_item,
           W_query, b_query, W_r1, b_r1, W_r2, b_r2, W_mq, b_mq,
           W_g, b_g, W_o, b_o, ln_g, ln_b):
    row2 = lambda v: v.reshape(1, -1)

    full = lambda shp: pl.BlockSpec(shp, lambda b: (0,) * len(shp))
    per_b3 = lambda d0, d1: pl.BlockSpec((BB, d0, d1), lambda b: (b, 0, 0))

    out_shapes = (
        jax.ShapeDtypeStruct((B, S, H), jnp.float32),
        jax.ShapeDtypeStruct((B, SLOTS, HCM), jnp.float32),
        jax.ShapeDtypeStruct((B, SLOTS, HCM), jnp.float32),
    )
    return pl.pallas_call(
        _fused_kernel,
        grid=(B // BB,),
        in_specs=[
            per_b3(S, H), per_b3(SLOTS, HCM), per_b3(SLOTS, HCM),
            full((H, HCM)), full((1, HCM)),
            full((H, H)), full((1, H)),
            full((HCM, 2 * HCM)), full((1, 2 * HCM)),
            full((2 * HCM, HCM)), full((1, HCM)),
            full((H, HCM)), full((1, HCM)),
            full((H, SLOTS)), full((HCM, SLOTS)), full((1, SLOTS)),
            full((H, H)), full((HCM, H)), full((1, H)),
            full((1, H)), full((1, H)),
        ],
        out_specs=(per_b3(S, H), per_b3(SLOTS, HCM), per_b3(SLOTS, HCM)),
        out_shape=out_shapes,
    )(hidden_states, fast_hcm_state, slow_hcm_state,
      W_item.astype(_BF), row2(b_item), W_query.astype(_BF), row2(b_query),
      W_r1.astype(_BF), row2(b_r1), W_r2.astype(_BF), row2(b_r2),
      W_mq.astype(_BF), row2(b_mq),
      W_g[:H, :SLOTS].astype(_BF), W_g[H:, :SLOTS].astype(_BF), row2(b_g[:SLOTS]),
      W_o[:H, :].astype(_BF), W_o[H:, :].astype(_BF), row2(b_o),
      row2(ln_g), row2(ln_b))


# speedup vs baseline: 1.1008x; 1.1008x over previous
import functools

import jax
import jax.numpy as jnp
from jax.experimental import pallas as pl

B, S, H = 8, 512, 1024
HCM = 512
SLOTS = 64
ALPHA = 0.1
GAMMA = 0.99
ETA = 0.05
BB = 2

_F32 = jnp.float32


def _fused_kernel(hs_ref, fast_ref, slow_ref,
                  w_item_ref, b_item_ref, w_query_ref, b_query_ref,
                  w_r1_ref, b_r1_ref, w_r2_ref, b_r2_ref,
                  w_mq_ref, b_mq_ref,
                  wg_h_ref, wg_r_ref, b_g_ref,
                  w_o_ref, b_o_ref,
                  ln_g_ref, ln_b_ref,
                  out_ref, newfast_ref, newslow_ref):
    R = range(BB)
    x = [hs_ref[i] for i in R]

    items = [jnp.dot(x[i], w_item_ref[...], preferred_element_type=_F32) + b_item_ref[...] for i in R]
    query = [jnp.dot(x[i], w_query_ref[...], preferred_element_type=_F32) + b_query_ref[...] for i in R]
    h1 = [jax.nn.gelu(jnp.dot(items[i], w_r1_ref[...], preferred_element_type=_F32) + b_r1_ref[...]) for i in R]
    ub = [jnp.dot(h1[i], w_r2_ref[...], preferred_element_type=_F32) + b_r2_ref[...] for i in R]
    q_mem = [jnp.dot(query[i], w_mq_ref[...], preferred_element_type=_F32) + b_mq_ref[...] for i in R]
    qk = [ub[i] * q_mem[i] for i in R]

    scale = 1.0 / jnp.sqrt(jnp.float32(HCM))
    mem = [jnp.concatenate([fast_ref[i], slow_ref[i]], axis=0) for i in R]
    scores = [jax.lax.dot_general(qk[i], mem[i], (((1,), (1,)), ((), ())),
                                  preferred_element_type=_F32) * scale for i in R]
    mx = [jnp.max(scores[i], axis=-1, keepdims=True) for i in R]
    p = [jnp.exp(scores[i] - mx[i]) for i in R]
    w = [p[i] / jnp.sum(p[i], axis=-1, keepdims=True) for i in R]
    retrieved = [jnp.dot(w[i], mem[i], preferred_element_type=_F32) * ub[i] for i in R]

    fg = [jax.nn.sigmoid(jnp.dot(x[i], wg_h_ref[...], preferred_element_type=_F32)
                         + jnp.dot(retrieved[i], wg_r_ref[...], preferred_element_type=_F32)
                         + b_g_ref[...]) for i in R]

    t = jax.lax.broadcasted_iota(jnp.int32, (S, 1), 0)
    cons = (t % 10) == 0
    e = jnp.where(cons, 1.0 - ETA, 1.0)
    colv = jax.lax.broadcasted_iota(jnp.int32, (1, S), 1)
    lower = (colv <= t).astype(_F32)
    nafter = (S - 1) // 10 - t // 10
    wv = jnp.where(cons, (ETA / (1.0 - ETA)) * jnp.exp(nafter.astype(_F32) * jnp.log(_F32(GAMMA))), 0.0)
    ncons = (S + 9) // 10

    for i in R:
        fast0 = fast_ref[i]
        slow0 = slow_ref[i]
        u = ALPHA * fg[i]
        logA = jnp.log((1.0 - u) * e)
        L = jnp.dot(lower, logA, preferred_element_type=_F32)
        Llast = L[S - 1:S, :]
        ue = u * e
        Cf = ue * jnp.exp(Llast - L)
        qv = wv * jnp.exp(L)
        wsum = jax.lax.dot_general(lower, qv, (((0,), (0,)), ((), ())),
                                   preferred_element_type=_F32)
        Cs = ue * wsum * jnp.exp(-L)
        plast_col = jnp.transpose(jnp.exp(Llast))
        w0_col = jnp.transpose(wsum[0:1, :])
        newfast_ref[i] = plast_col * fast0 + jax.lax.dot_general(
            Cf, items[i], (((0,), (0,)), ((), ())), preferred_element_type=_F32)
        newslow_ref[i] = (GAMMA ** ncons) * slow0 + w0_col * fast0 + jax.lax.dot_general(
            Cs, items[i], (((0,), (0,)), ((), ())), preferred_element_type=_F32)

    for i in R:
        out = (jnp.dot(query[i], w_o_ref[0:H, :], preferred_element_type=_F32)
               + jnp.dot(retrieved[i], w_o_ref[H:H + HCM, :], preferred_element_type=_F32)
               + b_o_ref[...])
        y = hs_ref[i] + out
        mu = jnp.mean(y, axis=-1, keepdims=True)
        var = jnp.mean((y - mu) ** 2, axis=-1, keepdims=True)
        out_ref[i] = (y - mu) / jnp.sqrt(var + 1e-5) * ln_g_ref[...] + ln_b_ref[...]


@functools.partial(jax.jit, static_argnames=())
def kernel(hidden_states, fast_hcm_state, slow_hcm_state, W_item, b_item,
           W_query, b_query, W_r1, b_r1, W_r2, b_r2, W_mq, b_mq,
           W_g, b_g, W_o, b_o, ln_g, ln_b):
    row2 = lambda v: v.reshape(1, -1)

    full = lambda shp: pl.BlockSpec(shp, lambda b: (0,) * len(shp))
    per_b3 = lambda d0, d1: pl.BlockSpec((BB, d0, d1), lambda b: (b, 0, 0))

    out_shapes = (
        jax.ShapeDtypeStruct((B, S, H), jnp.float32),
        jax.ShapeDtypeStruct((B, SLOTS, HCM), jnp.float32),
        jax.ShapeDtypeStruct((B, SLOTS, HCM), jnp.float32),
    )
    return pl.pallas_call(
        _fused_kernel,
        grid=(B // BB,),
        in_specs=[
            per_b3(S, H), per_b3(SLOTS, HCM), per_b3(SLOTS, HCM),
            full((H, HCM)), full((1, HCM)),
            full((H, H)), full((1, H)),
            full((HCM, 2 * HCM)), full((1, 2 * HCM)),
            full((2 * HCM, HCM)), full((1, HCM)),
            full((H, HCM)), full((1, HCM)),
            full((H, SLOTS)), full((HCM, SLOTS)), full((1, SLOTS)),
            full((H + HCM, H)), full((1, H)),
            full((1, H)), full((1, H)),
        ],
        out_specs=(per_b3(S, H), per_b3(SLOTS, HCM), per_b3(SLOTS, HCM)),
        out_shape=out_shapes,
    )(hidden_states, fast_hcm_state, slow_hcm_state,
      W_item, row2(b_item), W_query, row2(b_query),
      W_r1, row2(b_r1), W_r2, row2(b_r2),
      W_mq, row2(b_mq),
      W_g[:H, :SLOTS], W_g[H:, :SLOTS], row2(b_g[:SLOTS]),
      W_o, row2(b_o),
      row2(ln_g), row2(ln_b))


# speedup vs baseline: 1.2150x; 1.1037x over previous
import functools

import jax
import jax.numpy as jnp
from jax.experimental import pallas as pl

B, S, H = 8, 512, 1024
HCM = 512
SLOTS = 64
ALPHA = 0.1
GAMMA = 0.99
ETA = 0.05
BB = 2

_F32 = jnp.float32


def _fused_kernel(hs_ref, fast_ref, slow_ref,
                  w_item_ref, b_item_ref, w_query_ref, b_query_ref,
                  w_r1_ref, b_r1_ref, w_r2_ref, b_r2_ref,
                  w_mq_ref, b_mq_ref,
                  w_g_ref, b_g_ref,
                  w_o_ref, b_o_ref,
                  ln_g_ref, ln_b_ref,
                  out_ref, newfast_ref, newslow_ref):
    R = range(BB)
    x = [hs_ref[i] for i in R]

    items = [jnp.dot(x[i], w_item_ref[...], preferred_element_type=_F32) + b_item_ref[...] for i in R]
    query = [jnp.dot(x[i], w_query_ref[...], preferred_element_type=_F32) + b_query_ref[...] for i in R]
    h1 = [jax.nn.gelu(jnp.dot(items[i], w_r1_ref[...], preferred_element_type=_F32) + b_r1_ref[...]) for i in R]
    ub = [jnp.dot(h1[i], w_r2_ref[...], preferred_element_type=_F32) + b_r2_ref[...] for i in R]
    q_mem = [jnp.dot(query[i], w_mq_ref[...], preferred_element_type=_F32) + b_mq_ref[...] for i in R]
    qk = [ub[i] * q_mem[i] for i in R]

    scale = 1.0 / jnp.sqrt(jnp.float32(HCM))
    mem = [jnp.concatenate([fast_ref[i], slow_ref[i]], axis=0) for i in R]
    scores = [jax.lax.dot_general(qk[i], mem[i], (((1,), (1,)), ((), ())),
                                  preferred_element_type=_F32) * scale for i in R]
    mx = [jnp.max(scores[i], axis=-1, keepdims=True) for i in R]
    p = [jnp.exp(scores[i] - mx[i]) for i in R]
    w = [p[i] / jnp.sum(p[i], axis=-1, keepdims=True) for i in R]
    retrieved = [jnp.dot(w[i], mem[i], preferred_element_type=_F32) * ub[i] for i in R]

    fg = [jax.nn.sigmoid(jnp.dot(x[i], w_g_ref[0:H, 0:SLOTS], preferred_element_type=_F32)
                         + jnp.dot(retrieved[i], w_g_ref[H:H + HCM, 0:SLOTS], preferred_element_type=_F32)
                         + b_g_ref[0:1, 0:SLOTS]) for i in R]

    t = jax.lax.broadcasted_iota(jnp.int32, (S, 1), 0)
    cons = (t % 10) == 0
    e = jnp.where(cons, 1.0 - ETA, 1.0)
    colv = jax.lax.broadcasted_iota(jnp.int32, (1, S), 1)
    lower = (colv <= t).astype(_F32)
    nafter = (S - 1) // 10 - t // 10
    wv = jnp.where(cons, (ETA / (1.0 - ETA)) * jnp.exp(nafter.astype(_F32) * jnp.log(_F32(GAMMA))), 0.0)
    ncons = (S + 9) // 10

    for i in R:
        fast0 = fast_ref[i]
        slow0 = slow_ref[i]
        u = ALPHA * fg[i]
        logA = jnp.log((1.0 - u) * e)
        L = jnp.dot(lower, logA, preferred_element_type=_F32)
        Llast = L[S - 1:S, :]
        ue = u * e
        Cf = ue * jnp.exp(Llast - L)
        qv = wv * jnp.exp(L)
        wsum = jax.lax.dot_general(lower, qv, (((0,), (0,)), ((), ())),
                                   preferred_element_type=_F32)
        Cs = ue * wsum * jnp.exp(-L)
        plast_col = jnp.transpose(jnp.exp(Llast))
        w0_col = jnp.transpose(wsum[0:1, :])
        newfast_ref[i] = plast_col * fast0 + jax.lax.dot_general(
            Cf, items[i], (((0,), (0,)), ((), ())), preferred_element_type=_F32)
        newslow_ref[i] = (GAMMA ** ncons) * slow0 + w0_col * fast0 + jax.lax.dot_general(
            Cs, items[i], (((0,), (0,)), ((), ())), preferred_element_type=_F32)

    for i in R:
        out = (jnp.dot(query[i], w_o_ref[0:H, :], preferred_element_type=_F32)
               + jnp.dot(retrieved[i], w_o_ref[H:H + HCM, :], preferred_element_type=_F32)
               + b_o_ref[...])
        y = hs_ref[i] + out
        mu = jnp.mean(y, axis=-1, keepdims=True)
        var = jnp.mean((y - mu) ** 2, axis=-1, keepdims=True)
        out_ref[i] = (y - mu) / jnp.sqrt(var + 1e-5) * ln_g_ref[...] + ln_b_ref[...]


@functools.partial(jax.jit, static_argnames=())
def kernel(hidden_states, fast_hcm_state, slow_hcm_state, W_item, b_item,
           W_query, b_query, W_r1, b_r1, W_r2, b_r2, W_mq, b_mq,
           W_g, b_g, W_o, b_o, ln_g, ln_b):
    row2 = lambda v: v.reshape(1, -1)

    full = lambda shp: pl.BlockSpec(shp, lambda b: (0,) * len(shp))
    per_b3 = lambda d0, d1: pl.BlockSpec((BB, d0, d1), lambda b: (b, 0, 0))

    out_shapes = (
        jax.ShapeDtypeStruct((B, S, H), jnp.float32),
        jax.ShapeDtypeStruct((B, SLOTS, HCM), jnp.float32),
        jax.ShapeDtypeStruct((B, SLOTS, HCM), jnp.float32),
    )
    return pl.pallas_call(
        _fused_kernel,
        grid=(B // BB,),
        in_specs=[
            per_b3(S, H), per_b3(SLOTS, HCM), per_b3(SLOTS, HCM),
            full((H, HCM)), full((1, HCM)),
            full((H, H)), full((1, H)),
            full((HCM, 2 * HCM)), full((1, 2 * HCM)),
            full((2 * HCM, HCM)), full((1, HCM)),
            full((H, HCM)), full((1, HCM)),
            full((H + HCM, 2 * SLOTS)), full((1, 2 * SLOTS)),
            full((H + HCM, H)), full((1, H)),
            full((1, H)), full((1, H)),
        ],
        out_specs=(per_b3(S, H), per_b3(SLOTS, HCM), per_b3(SLOTS, HCM)),
        out_shape=out_shapes,
    )(hidden_states, fast_hcm_state, slow_hcm_state,
      W_item, row2(b_item), W_query, row2(b_query),
      W_r1, row2(b_r1), W_r2, row2(b_r2),
      W_mq, row2(b_mq),
      W_g, row2(b_g),
      W_o, row2(b_o),
      row2(ln_g), row2(ln_b))


# speedup vs baseline: 1.2330x; 1.0148x over previous
import functools

import jax
import jax.numpy as jnp
from jax.experimental import pallas as pl

B, S, H = 8, 512, 1024
HCM = 512
SLOTS = 64
ALPHA = 0.1
GAMMA = 0.99
ETA = 0.05
BB = 2

_F32 = jnp.float32


def _fused_kernel(hs_ref, fast_ref, slow_ref,
                  w_item_ref, b_item_ref, w_query_ref, b_query_ref,
                  w_r1_ref, b_r1_ref, w_r2_ref, b_r2_ref,
                  w_mq_ref, b_mq_ref,
                  w_g_ref, b_g_ref,
                  w_o_ref, b_o_ref,
                  ln_g_ref, ln_b_ref,
                  out_ref, newfast_ref, newslow_ref):
    R = range(BB)
    NC = BB * 2
    CH = S // 2
    C = range(NC)
    ex = lambda c: c // 2
    xc = [hs_ref[ex(c)][(c % 2) * CH:(c % 2) * CH + CH] for c in C]

    items_c = [jnp.dot(xc[c], w_item_ref[...], preferred_element_type=_F32) + b_item_ref[...] for c in C]
    query_c = [jnp.dot(xc[c], w_query_ref[...], preferred_element_type=_F32) + b_query_ref[...] for c in C]
    h1_c = [jax.nn.gelu(jnp.dot(items_c[c], w_r1_ref[...], preferred_element_type=_F32) + b_r1_ref[...]) for c in C]
    ub_c = [jnp.dot(h1_c[c], w_r2_ref[...], preferred_element_type=_F32) + b_r2_ref[...] for c in C]
    q_mem_c = [jnp.dot(query_c[c], w_mq_ref[...], preferred_element_type=_F32) + b_mq_ref[...] for c in C]
    qk_c = [ub_c[c] * q_mem_c[c] for c in C]

    scale = 1.0 / jnp.sqrt(jnp.float32(HCM))
    mem = [jnp.concatenate([fast_ref[i], slow_ref[i]], axis=0) for i in R]
    scores_c = [jax.lax.dot_general(qk_c[c], mem[ex(c)], (((1,), (1,)), ((), ())),
                                    preferred_element_type=_F32) * scale for c in C]
    mx = [jnp.max(scores_c[c], axis=-1, keepdims=True) for c in C]
    p = [jnp.exp(scores_c[c] - mx[c]) for c in C]
    w = [p[c] / jnp.sum(p[c], axis=-1, keepdims=True) for c in C]
    retr_c = [jnp.dot(w[c], mem[ex(c)], preferred_element_type=_F32) * ub_c[c] for c in C]

    fg_c = [jax.nn.sigmoid(jnp.dot(xc[c], w_g_ref[0:H, 0:SLOTS], preferred_element_type=_F32)
                           + jnp.dot(retr_c[c], w_g_ref[H:H + HCM, 0:SLOTS], preferred_element_type=_F32)
                           + b_g_ref[0:1, 0:SLOTS]) for c in C]
    fg = [jnp.concatenate([fg_c[2 * i], fg_c[2 * i + 1]], axis=0) for i in R]
    items = [jnp.concatenate([items_c[2 * i], items_c[2 * i + 1]], axis=0) for i in R]

    t = jax.lax.broadcasted_iota(jnp.int32, (S, 1), 0)
    cons = (t % 10) == 0
    e = jnp.where(cons, 1.0 - ETA, 1.0)
    colv = jax.lax.broadcasted_iota(jnp.int32, (1, S), 1)
    lower = (colv <= t).astype(_F32)
    nafter = (S - 1) // 10 - t // 10
    wv = jnp.where(cons, (ETA / (1.0 - ETA)) * jnp.exp(nafter.astype(_F32) * jnp.log(_F32(GAMMA))), 0.0)
    ncons = (S + 9) // 10

    for i in R:
        fast0 = fast_ref[i]
        slow0 = slow_ref[i]
        u = ALPHA * fg[i]
        logA = jnp.log((1.0 - u) * e)
        L = jnp.dot(lower, logA, preferred_element_type=_F32)
        Llast = L[S - 1:S, :]
        ue = u * e
        Cf = ue * jnp.exp(Llast - L)
        qv = wv * jnp.exp(L)
        wsum = jax.lax.dot_general(lower, qv, (((0,), (0,)), ((), ())),
                                   preferred_element_type=_F32)
        Cs = ue * wsum * jnp.exp(-L)
        plast_col = jnp.transpose(jnp.exp(Llast))
        w0_col = jnp.transpose(wsum[0:1, :])
        newfast_ref[i] = plast_col * fast0 + jax.lax.dot_general(
            Cf, items[i], (((0,), (0,)), ((), ())), preferred_element_type=_F32)
        newslow_ref[i] = (GAMMA ** ncons) * slow0 + w0_col * fast0 + jax.lax.dot_general(
            Cs, items[i], (((0,), (0,)), ((), ())), preferred_element_type=_F32)

    for c in C:
        out = (jnp.dot(query_c[c], w_o_ref[0:H, :], preferred_element_type=_F32)
               + jnp.dot(retr_c[c], w_o_ref[H:H + HCM, :], preferred_element_type=_F32)
               + b_o_ref[...])
        y = xc[c] + out
        mu = jnp.mean(y, axis=-1, keepdims=True)
        var = jnp.mean((y - mu) ** 2, axis=-1, keepdims=True)
        out_ref[ex(c), (c % 2) * CH:(c % 2) * CH + CH, :] = (
            (y - mu) / jnp.sqrt(var + 1e-5) * ln_g_ref[...] + ln_b_ref[...])


@functools.partial(jax.jit, static_argnames=())
def kernel(hidden_states, fast_hcm_state, slow_hcm_state, W_item, b_item,
           W_query, b_query, W_r1, b_r1, W_r2, b_r2, W_mq, b_mq,
           W_g, b_g, W_o, b_o, ln_g, ln_b):
    row2 = lambda v: v.reshape(1, -1)

    full = lambda shp: pl.BlockSpec(shp, lambda b: (0,) * len(shp))
    per_b3 = lambda d0, d1: pl.BlockSpec((BB, d0, d1), lambda b: (b, 0, 0))

    out_shapes = (
        jax.ShapeDtypeStruct((B, S, H), jnp.float32),
        jax.ShapeDtypeStruct((B, SLOTS, HCM), jnp.float32),
        jax.ShapeDtypeStruct((B, SLOTS, HCM), jnp.float32),
    )
    return pl.pallas_call(
        _fused_kernel,
        grid=(B // BB,),
        in_specs=[
            per_b3(S, H), per_b3(SLOTS, HCM), per_b3(SLOTS, HCM),
            full((H, HCM)), full((1, HCM)),
            full((H, H)), full((1, H)),
            full((HCM, 2 * HCM)), full((1, 2 * HCM)),
            full((2 * HCM, HCM)), full((1, HCM)),
            full((H, HCM)), full((1, HCM)),
            full((H + HCM, 2 * SLOTS)), full((1, 2 * SLOTS)),
            full((H + HCM, H)), full((1, H)),
            full((1, H)), full((1, H)),
        ],
        out_specs=(per_b3(S, H), per_b3(SLOTS, HCM), per_b3(SLOTS, HCM)),
        out_shape=out_shapes,
    )(hidden_states, fast_hcm_state, slow_hcm_state,
      W_item, row2(b_item), W_query, row2(b_query),
      W_r1, row2(b_r1), W_r2, row2(b_r2),
      W_mq, row2(b_mq),
      W_g, row2(b_g),
      W_o, row2(b_o),
      row2(ln_g), row2(ln_b))


# speedup vs baseline: 1.2355x; 1.0021x over previous
import functools

import jax
import jax.numpy as jnp
from jax.experimental import pallas as pl

B, S, H = 8, 512, 1024
HCM = 512
SLOTS = 64
ALPHA = 0.1
GAMMA = 0.99
ETA = 0.05
BB = 2

_F32 = jnp.float32


def _fused_kernel(hs_ref, fast_ref, slow_ref,
                  w_item_ref, b_item_ref, w_query_ref, b_query_ref,
                  w_r1_ref, b_r1_ref, w_r2_ref, b_r2_ref,
                  w_mq_ref, b_mq_ref,
                  w_g_ref, b_g_ref,
                  w_o_ref, b_o_ref,
                  ln_g_ref, ln_b_ref,
                  out_ref, newfast_ref, newslow_ref):
    R = range(BB)
    CPE = 2
    NC = BB * CPE
    CH = S // CPE
    C = range(NC)
    ex = lambda c: c // CPE
    xc = [hs_ref[ex(c)][(c % CPE) * CH:(c % CPE) * CH + CH] for c in C]

    items_c = [jnp.dot(xc[c], w_item_ref[...], preferred_element_type=_F32) + b_item_ref[...] for c in C]
    query_c = [jnp.dot(xc[c], w_query_ref[...], preferred_element_type=_F32) + b_query_ref[...] for c in C]
    h1_c = [jax.nn.gelu(jnp.dot(items_c[c], w_r1_ref[...], preferred_element_type=_F32) + b_r1_ref[...]) for c in C]
    ub_c = [jnp.dot(h1_c[c], w_r2_ref[...], preferred_element_type=_F32) + b_r2_ref[...] for c in C]
    q_mem_c = [jnp.dot(query_c[c], w_mq_ref[...], preferred_element_type=_F32) + b_mq_ref[...] for c in C]
    qk_c = [ub_c[c] * q_mem_c[c] for c in C]

    scale = 1.0 / jnp.sqrt(jnp.float32(HCM))
    mem = [jnp.concatenate([fast_ref[i], slow_ref[i]], axis=0) for i in R]
    scores_c = [jax.lax.dot_general(qk_c[c], mem[ex(c)], (((1,), (1,)), ((), ())),
                                    preferred_element_type=_F32) * scale for c in C]
    mx = [jnp.max(scores_c[c], axis=-1, keepdims=True) for c in C]
    p = [jnp.exp(scores_c[c] - mx[c]) for c in C]
    w = [p[c] / jnp.sum(p[c], axis=-1, keepdims=True) for c in C]
    retr_c = [jnp.dot(w[c], mem[ex(c)], preferred_element_type=_F32) * ub_c[c] for c in C]

    fg_c = [jax.nn.sigmoid(jnp.dot(xc[c], w_g_ref[0:H, 0:SLOTS], preferred_element_type=_F32)
                           + jnp.dot(retr_c[c], w_g_ref[H:H + HCM, 0:SLOTS], preferred_element_type=_F32)
                           + b_g_ref[0:1, 0:SLOTS]) for c in C]
    fg = [jnp.concatenate(fg_c[CPE * i:CPE * i + CPE], axis=0) for i in R]
    items = [jnp.concatenate(items_c[CPE * i:CPE * i + CPE], axis=0) for i in R]

    t = jax.lax.broadcasted_iota(jnp.int32, (S, 1), 0)
    cons = (t % 10) == 0
    e = jnp.where(cons, 1.0 - ETA, 1.0)
    colv = jax.lax.broadcasted_iota(jnp.int32, (1, S), 1)
    lower = (colv <= t).astype(_F32)
    nafter = (S - 1) // 10 - t // 10
    wv = jnp.where(cons, (ETA / (1.0 - ETA)) * jnp.exp(nafter.astype(_F32) * jnp.log(_F32(GAMMA))), 0.0)
    ncons = (S + 9) // 10

    for i in R:
        fast0 = fast_ref[i]
        slow0 = slow_ref[i]
        u = ALPHA * fg[i]
        logA = jnp.log((1.0 - u) * e)
        L = jnp.dot(lower, logA, preferred_element_type=_F32)
        Llast = L[S - 1:S, :]
        ue = u * e
        Cf = ue * jnp.exp(Llast - L)
        qv = wv * jnp.exp(L)
        wsum = jax.lax.dot_general(lower, qv, (((0,), (0,)), ((), ())),
                                   preferred_element_type=_F32)
        Cs = ue * wsum * jnp.exp(-L)
        plast_col = jnp.transpose(jnp.exp(Llast))
        w0_col = jnp.transpose(wsum[0:1, :])
        newfast_ref[i] = plast_col * fast0 + jax.lax.dot_general(
            Cf, items[i], (((0,), (0,)), ((), ())), preferred_element_type=_F32)
        newslow_ref[i] = (GAMMA ** ncons) * slow0 + w0_col * fast0 + jax.lax.dot_general(
            Cs, items[i], (((0,), (0,)), ((), ())), preferred_element_type=_F32)

    for c in C:
        out = (jnp.dot(query_c[c], w_o_ref[0:H, :], preferred_element_type=_F32)
               + jnp.dot(retr_c[c], w_o_ref[H:H + HCM, :], preferred_element_type=_F32)
               + b_o_ref[...])
        y = xc[c] + out
        mu = jnp.mean(y, axis=-1, keepdims=True)
        var = jnp.mean((y - mu) ** 2, axis=-1, keepdims=True)
        out_ref[ex(c), (c % CPE) * CH:(c % CPE) * CH + CH, :] = (
            (y - mu) / jnp.sqrt(var + 1e-5) * ln_g_ref[...] + ln_b_ref[...])


@functools.partial(jax.jit, static_argnames=())
def kernel(hidden_states, fast_hcm_state, slow_hcm_state, W_item, b_item,
           W_query, b_query, W_r1, b_r1, W_r2, b_r2, W_mq, b_mq,
           W_g, b_g, W_o, b_o, ln_g, ln_b):
    row2 = lambda v: v.reshape(1, -1)

    full = lambda shp: pl.BlockSpec(shp, lambda b: (0,) * len(shp))
    per_b3 = lambda d0, d1: pl.BlockSpec((BB, d0, d1), lambda b: (b, 0, 0))

    out_shapes = (
        jax.ShapeDtypeStruct((B, S, H), jnp.float32),
        jax.ShapeDtypeStruct((B, SLOTS, HCM), jnp.float32),
        jax.ShapeDtypeStruct((B, SLOTS, HCM), jnp.float32),
    )
    return pl.pallas_call(
        _fused_kernel,
        grid=(B // BB,),
        in_specs=[
            per_b3(S, H), per_b3(SLOTS, HCM), per_b3(SLOTS, HCM),
            full((H, HCM)), full((1, HCM)),
            full((H, H)), full((1, H)),
            full((HCM, 2 * HCM)), full((1, 2 * HCM)),
            full((2 * HCM, HCM)), full((1, HCM)),
            full((H, HCM)), full((1, HCM)),
            full((H + HCM, 2 * SLOTS)), full((1, 2 * SLOTS)),
            full((H + HCM, H)), full((1, H)),
            full((1, H)), full((1, H)),
        ],
        out_specs=(per_b3(S, H), per_b3(SLOTS, HCM), per_b3(SLOTS, HCM)),
        out_shape=out_shapes,
    )(hidden_states, fast_hcm_state, slow_hcm_state,
      W_item, row2(b_item), W_query, row2(b_query),
      W_r1, row2(b_r1), W_r2, row2(b_r2),
      W_mq, row2(b_mq),
      W_g, row2(b_g),
      W_o, row2(b_o),
      row2(ln_g), row2(ln_b))
